# baseline (speedup 1.0000x reference)
import functools

import jax
import jax.numpy as jnp
from jax import lax
from jax.experimental import pallas as pl
from jax.experimental.pallas import tpu as pltpu
from jax.experimental.pallas import tpu_sc as plsc

N = 4096
D = 256
K = 8192
P = 48

N_TILE = 512
K_TILE = 1024
N_GRID = N // N_TILE
K_GRID = K // K_TILE


def _encode_quantize_body(p_ref, we_ref, eb_ref, cbt_ref, oi_ref,
                          z_ref, idx_ref, loss_ref,
                          maxval_ref, argid_ref, cnh_ref, lidx_ref):
    n = pl.program_id(0)
    k = pl.program_id(1)

    @pl.when(n == 0)
    def _cnorm():
        cbt = cbt_ref[...]
        cnh_ref[0, pl.ds(k * K_TILE, K_TILE)] = 0.5 * jnp.sum(cbt * cbt, axis=0)

    @pl.when(k == 0)
    def _init():
        z = jnp.dot(p_ref[...], we_ref[...], preferred_element_type=jnp.float32)
        z = jnp.maximum(z + eb_ref[...], 0.0)
        z_ref[...] = z
        maxval_ref[...] = jnp.full((N_TILE, 1), -jnp.inf, dtype=jnp.float32)
        argid_ref[...] = jnp.zeros((N_TILE, 1), dtype=jnp.int32)

    cnh = cnh_ref[0, pl.ds(k * K_TILE, K_TILE)][None, :]
    t = jnp.dot(z_ref[...], cbt_ref[...],
                preferred_element_type=jnp.float32) - cnh
    rowmax = jnp.max(t, axis=1, keepdims=True)
    mask = jnp.where(t == rowmax, 1.0, 0.0)
    cs = jnp.dot(mask, oi_ref[...], preferred_element_type=jnp.float32)
    lidx_ref[...] = (4.0 * cs[:, 1:2] + cs[:, 2:3]).astype(jnp.int32)

    @pl.when(jnp.any(cs[:, 0:1] > 1.5))
    def _tie_fallback():
        iota = lax.broadcasted_iota(jnp.int32, t.shape, 1)
        lidx_ref[...] = jnp.min(jnp.where(t == rowmax, iota, K),
                                axis=1, keepdims=True)

    better = rowmax > maxval_ref[...]
    argid_ref[...] = jnp.where(better, lidx_ref[...] + k * K_TILE,
                               argid_ref[...])
    maxval_ref[...] = jnp.where(better, rowmax, maxval_ref[...])

    @pl.when(k == K_GRID - 1)
    def _finish():
        idx_ref[...] = argid_ref[...]
        z = z_ref[...]
        part = jnp.sum(z * z) - 2.0 * jnp.sum(maxval_ref[...])
        prev = jnp.where(n == 0, 0.0, loss_ref[0, 0])
        tot = prev + part
        scale = jnp.where(n == N_GRID - 1, 1.25 / (N * D), 1.0)
        loss_ref[0, 0] = tot * scale


def _encode_quantize(patches, We, enc_b, codebook_t):
    ar = jnp.arange(K_TILE, dtype=jnp.int32)
    ones_iota = jnp.concatenate(
        [jnp.ones((K_TILE, 1), jnp.float32),
         (ar // 4).astype(jnp.float32)[:, None],
         (ar % 4).astype(jnp.float32)[:, None],
         jnp.zeros((K_TILE, 5), jnp.float32)], axis=1)
    return pl.pallas_call(
        _encode_quantize_body,
        grid=(N_GRID, K_GRID),
        in_specs=[
            pl.BlockSpec((N_TILE, P), lambda n, k: (n, 0)),
            pl.BlockSpec((P, D), lambda n, k: (0, 0)),
            pl.BlockSpec((1, D), lambda n, k: (0, 0)),
            pl.BlockSpec((D, K_TILE), lambda n, k: (0, k)),
            pl.BlockSpec((K_TILE, 8), lambda n, k: (0, 0)),
        ],
        out_specs=[
            pl.BlockSpec((N_TILE, D), lambda n, k: (n, 0)),
            pl.BlockSpec((N_TILE, 1), lambda n, k: (n, 0)),
            pl.BlockSpec((1, 1), lambda n, k: (0, 0),
                         memory_space=pltpu.SMEM),
        ],
        out_shape=[
            jax.ShapeDtypeStruct((N, D), jnp.float32),
            jax.ShapeDtypeStruct((N, 1), jnp.int32),
            jax.ShapeDtypeStruct((1, 1), jnp.float32),
        ],
        scratch_shapes=[
            pltpu.VMEM((N_TILE, 1), jnp.float32),
            pltpu.VMEM((N_TILE, 1), jnp.int32),
            pltpu.VMEM((1, K), jnp.float32),
            pltpu.VMEM((N_TILE, 1), jnp.int32),
        ],
    )(patches, We, enc_b, codebook_t, ones_iota)


@functools.lru_cache(maxsize=1)
def _make_sc_gather():
    info = plsc.get_sparse_core_info()
    nw = info.num_cores * info.num_subcores
    b_per_w = N // nw
    mesh = plsc.VectorSubcoreMesh(core_axis_name="c", subcore_axis_name="s")

    @functools.partial(
        pl.kernel, mesh=mesh,
        out_type=jax.ShapeDtypeStruct((N, D), jnp.float32),
        scratch_types=[
            pltpu.VMEM((b_per_w,), jnp.int32),
            pltpu.VMEM((b_per_w, D), jnp.float32),
            pltpu.SemaphoreType.DMA,
        ],
    )
    def gather_kernel(table_hbm, idx_hbm, out_hbm, idx_v, rows_v, sem):
        wid = lax.axis_index("s") * info.num_cores + lax.axis_index("c")
        base = wid * b_per_w
        pltpu.sync_copy(idx_hbm.at[pl.ds(base, b_per_w)], idx_v)
        pltpu.async_copy(table_hbm.at[idx_v], rows_v, sem).wait()
        pltpu.sync_copy(rows_v, out_hbm.at[pl.ds(base, b_per_w)])

    return gather_kernel


def _decode_body(q_ref, wd_ref, bias_ref, out_ref):
    out_ref[...] = jnp.dot(q_ref[...], wd_ref[...],
                           preferred_element_type=jnp.float32) + bias_ref[...]


def _decode(q, Wd, bias_patch):
    return pl.pallas_call(
        _decode_body,
        grid=(N_GRID,),
        in_specs=[
            pl.BlockSpec((N_TILE, D), lambda n: (n, 0)),
            pl.BlockSpec((D, P), lambda n: (0, 0)),
            pl.BlockSpec((1, P), lambda n: (0, 0)),
        ],
        out_specs=pl.BlockSpec((N_TILE, P), lambda n: (n, 0)),
        out_shape=jax.ShapeDtypeStruct((N, P), jnp.float32),
    )(q, Wd, bias_patch)


def kernel(x, enc_W, enc_b, codebook, dec_W, dec_b):
    B = x.shape[0]
    patches = x.reshape(B, 3, 32, 4, 32, 4).transpose(0, 2, 4, 1, 3, 5).reshape(N, P)
    We = enc_W.reshape(D, P).T
    Wd = dec_W[:, :, ::-1, ::-1].transpose(1, 0, 2, 3).reshape(D, P)
    bias_patch = jnp.repeat(dec_b, 16).reshape(1, P)

    z, idx, loss = _encode_quantize(patches, We, enc_b.reshape(1, D),
                                    codebook.T)
    q = _make_sc_gather()(codebook, idx.reshape(N))
    recon_p = _decode(q, Wd, bias_patch)
    recon = (recon_p.reshape(B, 32, 32, 3, 4, 4)
             .transpose(0, 3, 1, 4, 2, 5).reshape(B, 3, 128, 128))
    return recon, loss[0, 0]


# speedup vs baseline: 1.2111x; 1.2111x over previous
import functools

import jax
import jax.numpy as jnp
from jax import lax
from jax.experimental import pallas as pl
from jax.experimental.pallas import tpu as pltpu
from jax.experimental.pallas import tpu_sc as plsc

N = 4096
D = 256
K = 8192
P = 48

N_TILE = 512
K_TILE = 512
N_GRID = N // N_TILE
K_GRID = K // K_TILE
SUB_UNROLL = 2


def _encode_quantize_body(p_ref, we_ref, eb_ref, cbt_ref,
                          z_ref, idx_ref, loss_ref, cnh_ref):
    n = pl.program_id(0)

    @pl.when(n == 0)
    def _cnorm():
        cbt = cbt_ref[...]
        cnh_ref[...] = 0.5 * jnp.sum(cbt * cbt, axis=0)[None, :]

    z = jnp.dot(p_ref[...], we_ref[...], preferred_element_type=jnp.float32)
    z = jnp.maximum(z + eb_ref[...], 0.0)
    z_ref[...] = z

    def sub(i, carry):
        maxval, argid = carry
        off = i * K_TILE
        cb = cbt_ref[:, pl.ds(off, K_TILE)]
        cn = cnh_ref[0, pl.ds(off, K_TILE)][None, :]
        t = jnp.dot(z, cb, preferred_element_type=jnp.float32) - cn
        rm = jnp.max(t, axis=1, keepdims=True)
        iota = lax.broadcasted_iota(jnp.int32, t.shape, 1)
        li = (jnp.min(jnp.where(t == rm, iota, K_TILE),
                      axis=1, keepdims=True) + off)
        better = rm > maxval
        return (jnp.where(better, rm, maxval),
                jnp.where(better, li, argid))

    maxval0 = jnp.full((N_TILE, 1), -jnp.inf, dtype=jnp.float32)
    argid0 = jnp.zeros((N_TILE, 1), dtype=jnp.int32)
    maxval, argid = lax.fori_loop(0, K_GRID, sub, (maxval0, argid0),
                                  unroll=SUB_UNROLL)

    idx_ref[...] = argid
    part = jnp.sum(z * z) - 2.0 * jnp.sum(maxval)
    prev = jnp.where(n == 0, 0.0, loss_ref[0, 0])
    tot = prev + part
    scale = jnp.where(n == N_GRID - 1, 1.25 / (N * D), 1.0)
    loss_ref[0, 0] = tot * scale


def _encode_quantize(patches, We, enc_b, codebook_t):
    return pl.pallas_call(
        _encode_quantize_body,
        grid=(N_GRID,),
        in_specs=[
            pl.BlockSpec((N_TILE, P), lambda n: (n, 0)),
            pl.BlockSpec((P, D), lambda n: (0, 0)),
            pl.BlockSpec((1, D), lambda n: (0, 0)),
            pl.BlockSpec((D, K), lambda n: (0, 0)),
        ],
        out_specs=[
            pl.BlockSpec((N_TILE, D), lambda n: (n, 0)),
            pl.BlockSpec((N_TILE, 1), lambda n: (n, 0)),
            pl.BlockSpec((1, 1), lambda n: (0, 0),
                         memory_space=pltpu.SMEM),
        ],
        out_shape=[
            jax.ShapeDtypeStruct((N, D), jnp.float32),
            jax.ShapeDtypeStruct((N, 1), jnp.int32),
            jax.ShapeDtypeStruct((1, 1), jnp.float32),
        ],
        scratch_shapes=[
            pltpu.VMEM((1, K), jnp.float32),
        ],
    )(patches, We, enc_b, codebook_t)


@functools.lru_cache(maxsize=1)
def _make_sc_gather():
    info = plsc.get_sparse_core_info()
    nw = info.num_cores * info.num_subcores
    b_per_w = N // nw
    mesh = plsc.VectorSubcoreMesh(core_axis_name="c", subcore_axis_name="s")

    @functools.partial(
        pl.kernel, mesh=mesh,
        out_type=jax.ShapeDtypeStruct((N, D), jnp.float32),
        scratch_types=[
            pltpu.VMEM((b_per_w,), jnp.int32),
            pltpu.VMEM((b_per_w, D), jnp.float32),
            pltpu.SemaphoreType.DMA,
        ],
    )
    def gather_kernel(table_hbm, idx_hbm, out_hbm, idx_v, rows_v, sem):
        wid = lax.axis_index("s") * info.num_cores + lax.axis_index("c")
        base = wid * b_per_w
        pltpu.sync_copy(idx_hbm.at[pl.ds(base, b_per_w)], idx_v)
        pltpu.async_copy(table_hbm.at[idx_v], rows_v, sem).wait()
        pltpu.sync_copy(rows_v, out_hbm.at[pl.ds(base, b_per_w)])

    return gather_kernel


def _decode_body(q_ref, wd_ref, bias_ref, out_ref):
    out_ref[...] = jnp.dot(q_ref[...], wd_ref[...],
                           preferred_element_type=jnp.float32) + bias_ref[...]


def _decode(q, Wd, bias_patch):
    return pl.pallas_call(
        _decode_body,
        grid=(N_GRID,),
        in_specs=[
            pl.BlockSpec((N_TILE, D), lambda n: (n, 0)),
            pl.BlockSpec((D, P), lambda n: (0, 0)),
            pl.BlockSpec((1, P), lambda n: (0, 0)),
        ],
        out_specs=pl.BlockSpec((N_TILE, P), lambda n: (n, 0)),
        out_shape=jax.ShapeDtypeStruct((N, P), jnp.float32),
    )(q, Wd, bias_patch)


def kernel(x, enc_W, enc_b, codebook, dec_W, dec_b):
    B = x.shape[0]
    patches = x.reshape(B, 3, 32, 4, 32, 4).transpose(0, 2, 4, 1, 3, 5).reshape(N, P)
    We = enc_W.reshape(D, P).T
    Wd = dec_W[:, :, ::-1, ::-1].transpose(1, 0, 2, 3).reshape(D, P)
    bias_patch = jnp.repeat(dec_b, 16).reshape(1, P)

    z, idx, loss = _encode_quantize(patches, We, enc_b.reshape(1, D),
                                    codebook.T)
    q = _make_sc_gather()(codebook, idx.reshape(N))
    recon_p = _decode(q, Wd, bias_patch)
    recon = (recon_p.reshape(B, 32, 32, 3, 4, 4)
             .transpose(0, 3, 1, 4, 2, 5).reshape(B, 3, 128, 128))
    return recon, loss[0, 0]


# speedup vs baseline: 1.2292x; 1.0149x over previous
import functools

import jax
import jax.numpy as jnp
from jax import lax
from jax.experimental import pallas as pl
from jax.experimental.pallas import tpu as pltpu
from jax.experimental.pallas import tpu_sc as plsc

N = 4096
D = 256
K = 8192
P = 48

N_TILE = 512
K_TILE = 512
N_GRID = N // N_TILE
K_GRID = K // K_TILE
SUB_UNROLL = 4


def _encode_quantize_body(p_ref, we_ref, eb_ref, cbt_ref,
                          z_ref, idx_ref, loss_ref, cnh_ref):
    n = pl.program_id(0)

    @pl.when(n == 0)
    def _cnorm():
        cbt = cbt_ref[...]
        cnh_ref[...] = 0.5 * jnp.sum(cbt * cbt, axis=0)[None, :]

    z = jnp.dot(p_ref[...], we_ref[...], preferred_element_type=jnp.float32)
    z = jnp.maximum(z + eb_ref[...], 0.0)
    z_ref[...] = z

    def sub(i, carry):
        maxval, argid = carry
        off = i * K_TILE
        cb = cbt_ref[:, pl.ds(off, K_TILE)]
        cn = cnh_ref[0, pl.ds(off, K_TILE)][None, :]
        t = jnp.dot(z, cb, preferred_element_type=jnp.float32) - cn
        rm = jnp.max(t, axis=1, keepdims=True)
        iota = lax.broadcasted_iota(jnp.int32, t.shape, 1)
        li = (jnp.min(jnp.where(t == rm, iota, K_TILE),
                      axis=1, keepdims=True) + off)
        better = rm > maxval
        return (jnp.where(better, rm, maxval),
                jnp.where(better, li, argid))

    maxval0 = jnp.full((N_TILE, 1), -jnp.inf, dtype=jnp.float32)
    argid0 = jnp.zeros((N_TILE, 1), dtype=jnp.int32)
    maxval, argid = lax.fori_loop(0, K_GRID, sub, (maxval0, argid0),
                                  unroll=SUB_UNROLL)

    idx_ref[...] = argid
    part = jnp.sum(z * z) - 2.0 * jnp.sum(maxval)
    prev = jnp.where(n == 0, 0.0, loss_ref[0, 0])
    tot = prev + part
    scale = jnp.where(n == N_GRID - 1, 1.25 / (N * D), 1.0)
    loss_ref[0, 0] = tot * scale


def _encode_quantize(patches, We, enc_b, codebook_t):
    return pl.pallas_call(
        _encode_quantize_body,
        grid=(N_GRID,),
        in_specs=[
            pl.BlockSpec((N_TILE, P), lambda n: (n, 0)),
            pl.BlockSpec((P, D), lambda n: (0, 0)),
            pl.BlockSpec((1, D), lambda n: (0, 0)),
            pl.BlockSpec((D, K), lambda n: (0, 0)),
        ],
        out_specs=[
            pl.BlockSpec((N_TILE, D), lambda n: (n, 0)),
            pl.BlockSpec((N_TILE, 1), lambda n: (n, 0)),
            pl.BlockSpec((1, 1), lambda n: (0, 0),
                         memory_space=pltpu.SMEM),
        ],
        out_shape=[
            jax.ShapeDtypeStruct((N, D), jnp.float32),
            jax.ShapeDtypeStruct((N, 1), jnp.int32),
            jax.ShapeDtypeStruct((1, 1), jnp.float32),
        ],
        scratch_shapes=[
            pltpu.VMEM((1, K), jnp.float32),
        ],
    )(patches, We, enc_b, codebook_t)


@functools.lru_cache(maxsize=1)
def _make_sc_gather():
    info = plsc.get_sparse_core_info()
    nw = info.num_cores * info.num_subcores
    b_per_w = N // nw
    mesh = plsc.VectorSubcoreMesh(core_axis_name="c", subcore_axis_name="s")

    @functools.partial(
        pl.kernel, mesh=mesh,
        out_type=jax.ShapeDtypeStruct((N, D), jnp.float32),
        scratch_types=[
            pltpu.VMEM((b_per_w,), jnp.int32),
            pltpu.VMEM((b_per_w, D), jnp.float32),
            pltpu.SemaphoreType.DMA,
        ],
    )
    def gather_kernel(table_hbm, idx_hbm, out_hbm, idx_v, rows_v, sem):
        wid = lax.axis_index("s") * info.num_cores + lax.axis_index("c")
        base = wid * b_per_w
        pltpu.sync_copy(idx_hbm.at[pl.ds(base, b_per_w)], idx_v)
        pltpu.async_copy(table_hbm.at[idx_v], rows_v, sem).wait()
        pltpu.sync_copy(rows_v, out_hbm.at[pl.ds(base, b_per_w)])

    return gather_kernel


def _decode_body(q_ref, wd_ref, bias_ref, out_ref):
    out_ref[...] = jnp.dot(q_ref[...], wd_ref[...],
                           preferred_element_type=jnp.float32) + bias_ref[...]


def _decode(q, Wd, bias_patch):
    return pl.pallas_call(
        _decode_body,
        grid=(N_GRID,),
        in_specs=[
            pl.BlockSpec((N_TILE, D), lambda n: (n, 0)),
            pl.BlockSpec((D, P), lambda n: (0, 0)),
            pl.BlockSpec((1, P), lambda n: (0, 0)),
        ],
        out_specs=pl.BlockSpec((N_TILE, P), lambda n: (n, 0)),
        out_shape=jax.ShapeDtypeStruct((N, P), jnp.float32),
    )(q, Wd, bias_patch)


def kernel(x, enc_W, enc_b, codebook, dec_W, dec_b):
    B = x.shape[0]
    patches = x.reshape(B, 3, 32, 4, 32, 4).transpose(0, 2, 4, 1, 3, 5).reshape(N, P)
    We = enc_W.reshape(D, P).T
    Wd = dec_W[:, :, ::-1, ::-1].transpose(1, 0, 2, 3).reshape(D, P)
    bias_patch = jnp.repeat(dec_b, 16).reshape(1, P)

    z, idx, loss = _encode_quantize(patches, We, enc_b.reshape(1, D),
                                    codebook.T)
    q = _make_sc_gather()(codebook, idx.reshape(N))
    recon_p = _decode(q, Wd, bias_patch)
    recon = (recon_p.reshape(B, 32, 32, 3, 4, 4)
             .transpose(0, 3, 1, 4, 2, 5).reshape(B, 3, 128, 128))
    return recon, loss[0, 0]


# speedup vs baseline: 1.3748x; 1.1185x over previous
import functools

import jax
import jax.numpy as jnp
from jax import lax
from jax.experimental import pallas as pl
from jax.experimental.pallas import tpu as pltpu
from jax.experimental.pallas import tpu_sc as plsc

N = 4096
D = 256
K = 8192
P = 48
P_PAD = 128

N_TILE = 512
K_TILE = 512
N_GRID = N // N_TILE
K_GRID = K // K_TILE
SUB_UNROLL = 4


def _encode_quantize_body(p_ref, wet_ref, eb_ref, cb_ref, wd_ref, bias_ref,
                          idx_ref, loss_ref, cw_ref, cnh_ref):
    n = pl.program_id(0)

    @pl.when(n == 0)
    def _once():
        for i in range(K_GRID):
            cb_i = cb_ref[pl.ds(i * K_TILE, K_TILE), :]
            cnh_ref[i] = 0.5 * jnp.sum(cb_i * cb_i, axis=1, keepdims=True)
        cw_ref[...] = (jnp.dot(cb_ref[...], wd_ref[...],
                               preferred_element_type=jnp.float32)
                       + bias_ref[...])

    pT = jnp.transpose(p_ref[...])
    zT = jnp.dot(wet_ref[...], pT, preferred_element_type=jnp.float32)
    zT = jnp.maximum(zT + eb_ref[...], 0.0)

    def sub(i, carry):
        maxval, argid = carry
        off = i * K_TILE
        cb_i = cb_ref[pl.ds(off, K_TILE), :]
        tT = (jnp.dot(cb_i, zT, preferred_element_type=jnp.float32)
              - cnh_ref[i])
        cm = jnp.max(tT, axis=0, keepdims=True)
        iota = lax.broadcasted_iota(jnp.int32, tT.shape, 0)
        li = (jnp.min(jnp.where(tT == cm, iota, K_TILE),
                      axis=0, keepdims=True) + off)
        better = cm > maxval
        return (jnp.where(better, cm, maxval),
                jnp.where(better, li, argid))

    maxval0 = jnp.full((1, N_TILE), -jnp.inf, dtype=jnp.float32)
    argid0 = jnp.zeros((1, N_TILE), dtype=jnp.int32)
    maxval, argid = lax.fori_loop(0, K_GRID, sub, (maxval0, argid0),
                                  unroll=SUB_UNROLL)

    idx_ref[0] = argid
    part = jnp.sum(zT * zT) - 2.0 * jnp.sum(maxval)
    prev = jnp.where(n == 0, 0.0, loss_ref[0, 0])
    tot = prev + part
    scale = jnp.where(n == N_GRID - 1, 1.25 / (N * D), 1.0)
    loss_ref[0, 0] = tot * scale


def _encode_quantize(patches, WeT, enc_b, codebook, Wd, bias_patch):
    return pl.pallas_call(
        _encode_quantize_body,
        grid=(N_GRID,),
        in_specs=[
            pl.BlockSpec((N_TILE, P), lambda n: (n, 0)),
            pl.BlockSpec((D, P), lambda n: (0, 0)),
            pl.BlockSpec((D, 1), lambda n: (0, 0)),
            pl.BlockSpec((K, D), lambda n: (0, 0)),
            pl.BlockSpec((D, P_PAD), lambda n: (0, 0)),
            pl.BlockSpec((1, P_PAD), lambda n: (0, 0)),
        ],
        out_specs=[
            pl.BlockSpec((1, 1, N_TILE), lambda n: (n, 0, 0)),
            pl.BlockSpec((1, 1), lambda n: (0, 0),
                         memory_space=pltpu.SMEM),
            pl.BlockSpec((K, P_PAD), lambda n: (0, 0)),
        ],
        out_shape=[
            jax.ShapeDtypeStruct((N_GRID, 1, N_TILE), jnp.int32),
            jax.ShapeDtypeStruct((1, 1), jnp.float32),
            jax.ShapeDtypeStruct((K, P_PAD), jnp.float32),
        ],
        scratch_shapes=[
            pltpu.VMEM((K_GRID, K_TILE, 1), jnp.float32),
        ],
    )(patches, WeT, enc_b, codebook, Wd, bias_patch)


@functools.lru_cache(maxsize=1)
def _make_sc_gather():
    info = plsc.get_sparse_core_info()
    nw = info.num_cores * info.num_subcores
    b_per_w = N // nw
    mesh = plsc.VectorSubcoreMesh(core_axis_name="c", subcore_axis_name="s")

    @functools.partial(
        pl.kernel, mesh=mesh,
        out_type=jax.ShapeDtypeStruct((N, P_PAD), jnp.float32),
        scratch_types=[
            pltpu.VMEM((b_per_w,), jnp.int32),
            pltpu.VMEM((b_per_w, P_PAD), jnp.float32),
            pltpu.SemaphoreType.DMA,
        ],
    )
    def gather_kernel(table_hbm, idx_hbm, out_hbm, idx_v, rows_v, sem):
        wid = lax.axis_index("s") * info.num_cores + lax.axis_index("c")
        base = wid * b_per_w
        pltpu.sync_copy(idx_hbm.at[pl.ds(base, b_per_w)], idx_v)
        pltpu.async_copy(table_hbm.at[idx_v], rows_v, sem).wait()
        pltpu.sync_copy(rows_v, out_hbm.at[pl.ds(base, b_per_w)])

    return gather_kernel


def kernel(x, enc_W, enc_b, codebook, dec_W, dec_b):
    B = x.shape[0]
    patches = x.reshape(B, 3, 32, 4, 32, 4).transpose(0, 2, 4, 1, 3, 5).reshape(N, P)
    WeT = enc_W.reshape(D, P)
    Wd = dec_W[:, :, ::-1, ::-1].transpose(1, 0, 2, 3).reshape(D, P)
    Wd = jnp.concatenate([Wd, jnp.zeros((D, P_PAD - P), Wd.dtype)], axis=1)
    bias_patch = jnp.concatenate(
        [jnp.repeat(dec_b, 16), jnp.zeros((P_PAD - P,), dec_b.dtype)]
    ).reshape(1, P_PAD)

    idx, loss, cw = _encode_quantize(patches, WeT, enc_b.reshape(D, 1),
                                     codebook, Wd, bias_patch)
    recon_p = _make_sc_gather()(cw, idx.reshape(N))[:, :P]
    recon = (recon_p.reshape(B, 32, 32, 3, 4, 4)
             .transpose(0, 3, 1, 4, 2, 5).reshape(B, 3, 128, 128))
    return recon, loss[0, 0]


# speedup vs baseline: 1.9202x; 1.3967x over previous
import functools

import jax
import jax.numpy as jnp
from jax import lax
from jax.experimental import pallas as pl
from jax.experimental.pallas import tpu as pltpu
from jax.experimental.pallas import tpu_sc as plsc

N = 4096
D = 256
K = 8192
P = 48
P_PAD = 128

N_TILE = 512
K_TILE = 512
N_GRID = N // N_TILE
K_GRID = K // K_TILE
SUB_UNROLL = 4


def _encode_quantize_body(x_ref, wet_ref, eb_ref, cb_ref, wd_ref, bias_ref,
                          idx_ref, loss_ref, cw_ref, cnh_ref):
    n = pl.program_id(0)

    @pl.when(n == 0)
    def _once():
        for i in range(K_GRID):
            cb_i = cb_ref[pl.ds(i * K_TILE, K_TILE), :]
            cnh_ref[i] = 0.5 * jnp.sum(cb_i * cb_i, axis=1, keepdims=True)
        cw_ref[...] = (jnp.dot(cb_ref[...], wd_ref[...],
                               preferred_element_type=jnp.float32)
                       + bias_ref[...])

    xb = x_ref[0, :, 0]
    pT = (xb.reshape(3, 16, 4, 32, 4)
            .transpose(0, 2, 4, 1, 3)
            .reshape(P, N_TILE))
    zT = jnp.dot(wet_ref[...], pT, preferred_element_type=jnp.float32)
    zT = jnp.maximum(zT + eb_ref[...], 0.0)

    def sub(i, carry):
        maxval, argid = carry
        off = i * K_TILE
        cb_i = cb_ref[pl.ds(off, K_TILE), :]
        tT = (jnp.dot(cb_i, zT, preferred_element_type=jnp.float32)
              - cnh_ref[i])
        cm = jnp.max(tT, axis=0, keepdims=True)
        iota = lax.broadcasted_iota(jnp.int32, tT.shape, 0)
        li = (jnp.min(jnp.where(tT == cm, iota, K_TILE),
                      axis=0, keepdims=True) + off)
        better = cm > maxval
        return (jnp.where(better, cm, maxval),
                jnp.where(better, li, argid))

    maxval0 = jnp.full((1, N_TILE), -jnp.inf, dtype=jnp.float32)
    argid0 = jnp.zeros((1, N_TILE), dtype=jnp.int32)
    maxval, argid = lax.fori_loop(0, K_GRID, sub, (maxval0, argid0),
                                  unroll=SUB_UNROLL)

    idx_ref[0] = argid
    part = jnp.sum(zT * zT) - 2.0 * jnp.sum(maxval)
    prev = jnp.where(n == 0, 0.0, loss_ref[0, 0])
    tot = prev + part
    scale = jnp.where(n == N_GRID - 1, 1.25 / (N * D), 1.0)
    loss_ref[0, 0] = tot * scale


def _encode_quantize(x5, WeT, enc_b, codebook, Wd, bias_patch):
    return pl.pallas_call(
        _encode_quantize_body,
        grid=(N_GRID,),
        in_specs=[
            pl.BlockSpec((1, 3, 1, 64, 128),
                         lambda n: (n // 2, 0, n % 2, 0, 0)),
            pl.BlockSpec((D, P), lambda n: (0, 0)),
            pl.BlockSpec((D, 1), lambda n: (0, 0)),
            pl.BlockSpec((K, D), lambda n: (0, 0)),
            pl.BlockSpec((D, P_PAD), lambda n: (0, 0)),
            pl.BlockSpec((1, P_PAD), lambda n: (0, 0)),
        ],
        out_specs=[
            pl.BlockSpec((1, 1, N_TILE), lambda n: (n, 0, 0)),
            pl.BlockSpec((1, 1), lambda n: (0, 0),
                         memory_space=pltpu.SMEM),
            pl.BlockSpec((K, P_PAD), lambda n: (0, 0)),
        ],
        out_shape=[
            jax.ShapeDtypeStruct((N_GRID, 1, N_TILE), jnp.int32),
            jax.ShapeDtypeStruct((1, 1), jnp.float32),
            jax.ShapeDtypeStruct((K, P_PAD), jnp.float32),
        ],
        scratch_shapes=[
            pltpu.VMEM((K_GRID, K_TILE, 1), jnp.float32),
        ],
    )(x5, WeT, enc_b, codebook, Wd, bias_patch)


@functools.lru_cache(maxsize=1)
def _make_sc_gather():
    info = plsc.get_sparse_core_info()
    nw = info.num_cores * info.num_subcores
    b_per_w = N // nw
    mesh = plsc.VectorSubcoreMesh(core_axis_name="c", subcore_axis_name="s")

    @functools.partial(
        pl.kernel, mesh=mesh,
        out_type=jax.ShapeDtypeStruct((N, P_PAD), jnp.float32),
        scratch_types=[
            pltpu.VMEM((b_per_w,), jnp.int32),
            pltpu.VMEM((b_per_w, P_PAD), jnp.float32),
            pltpu.SemaphoreType.DMA,
        ],
    )
    def gather_kernel(table_hbm, idx_hbm, out_hbm, idx_v, rows_v, sem):
        wid = lax.axis_index("s") * info.num_cores + lax.axis_index("c")
        base = wid * b_per_w
        pltpu.sync_copy(idx_hbm.at[pl.ds(base, b_per_w)], idx_v)
        pltpu.async_copy(table_hbm.at[idx_v], rows_v, sem).wait()
        pltpu.sync_copy(rows_v, out_hbm.at[pl.ds(base, b_per_w)])

    return gather_kernel


def _unpatchify_body(r_ref, o_ref):
    rp = r_ref[...][:, :P]
    o_ref[0, :, 0] = (rp.reshape(16, 32, 3, 4, 4)
                        .transpose(2, 0, 3, 1, 4)
                        .reshape(3, 64, 128))


def _unpatchify(rp):
    return pl.pallas_call(
        _unpatchify_body,
        grid=(N_GRID,),
        in_specs=[pl.BlockSpec((N_TILE, P_PAD), lambda n: (n, 0))],
        out_specs=pl.BlockSpec((1, 3, 1, 64, 128),
                               lambda n: (n // 2, 0, n % 2, 0, 0)),
        out_shape=jax.ShapeDtypeStruct((4, 3, 2, 64, 128), jnp.float32),
    )(rp)


def kernel(x, enc_W, enc_b, codebook, dec_W, dec_b):
    B = x.shape[0]
    x5 = x.reshape(B, 3, 2, 64, 128)
    WeT = enc_W.reshape(D, P)
    Wd = dec_W[:, :, ::-1, ::-1].transpose(1, 0, 2, 3).reshape(D, P)
    Wd = jnp.concatenate([Wd, jnp.zeros((D, P_PAD - P), Wd.dtype)], axis=1)
    bias_patch = jnp.concatenate(
        [jnp.repeat(dec_b, 16), jnp.zeros((P_PAD - P,), dec_b.dtype)]
    ).reshape(1, P_PAD)

    idx, loss, cw = _encode_quantize(x5, WeT, enc_b.reshape(D, 1),
                                     codebook, Wd, bias_patch)
    recon_p = _make_sc_gather()(cw, idx.reshape(N))
    recon = _unpatchify(recon_p).reshape(B, 3, 128, 128)
    return recon, loss[0, 0]


# speedup vs baseline: 1.9897x; 1.0362x over previous
import functools

import jax
import jax.numpy as jnp
from jax import lax
from jax.experimental import pallas as pl
from jax.experimental.pallas import tpu as pltpu
from jax.experimental.pallas import tpu_sc as plsc

N = 4096
D = 256
K = 8192
P = 48
P_PAD = 128

N_TILE = 512
K_TILE = 512
N_GRID = N // N_TILE
K_GRID = K // K_TILE
SUB_UNROLL = 8


def _encode_quantize_body(x_ref, wet_ref, eb_ref, cb_ref, wd_ref, bias_ref,
                          idx_ref, loss_ref, cw_ref, cnh_ref):
    n = pl.program_id(0)

    @pl.when(n == 0)
    def _once():
        for i in range(K_GRID):
            cb_i = cb_ref[pl.ds(i * K_TILE, K_TILE), :]
            cnh_ref[i] = 0.5 * jnp.sum(cb_i * cb_i, axis=1, keepdims=True)
        cw_ref[...] = (jnp.dot(cb_ref[...], wd_ref[...],
                               preferred_element_type=jnp.float32)
                       + bias_ref[...])

    xb = x_ref[0, :, 0]
    pT = (xb.reshape(3, 16, 4, 32, 4)
            .transpose(0, 2, 4, 1, 3)
            .reshape(P, N_TILE))
    zT = jnp.dot(wet_ref[...], pT, preferred_element_type=jnp.float32)
    zT = jnp.maximum(zT + eb_ref[...], 0.0)

    def sub(i, carry):
        maxval, argid = carry
        off = i * K_TILE
        cb_i = cb_ref[pl.ds(off, K_TILE), :]
        tT = (jnp.dot(cb_i, zT, preferred_element_type=jnp.float32)
              - cnh_ref[i])
        cm = jnp.max(tT, axis=0, keepdims=True)
        iota = lax.broadcasted_iota(jnp.int32, tT.shape, 0)
        li = (jnp.min(jnp.where(tT == cm, iota, K_TILE),
                      axis=0, keepdims=True) + off)
        better = cm > maxval
        return (jnp.where(better, cm, maxval),
                jnp.where(better, li, argid))

    maxval0 = jnp.full((1, N_TILE), -jnp.inf, dtype=jnp.float32)
    argid0 = jnp.zeros((1, N_TILE), dtype=jnp.int32)
    maxval, argid = lax.fori_loop(0, K_GRID, sub, (maxval0, argid0),
                                  unroll=SUB_UNROLL)

    idx_ref[0] = argid
    part = jnp.sum(zT * zT) - 2.0 * jnp.sum(maxval)
    prev = jnp.where(n == 0, 0.0, loss_ref[0, 0])
    tot = prev + part
    scale = jnp.where(n == N_GRID - 1, 1.25 / (N * D), 1.0)
    loss_ref[0, 0] = tot * scale


def _encode_quantize(x5, WeT, enc_b, codebook, Wd, bias_patch):
    return pl.pallas_call(
        _encode_quantize_body,
        grid=(N_GRID,),
        in_specs=[
            pl.BlockSpec((1, 3, 1, 64, 128),
                         lambda n: (n // 2, 0, n % 2, 0, 0)),
            pl.BlockSpec((D, P), lambda n: (0, 0)),
            pl.BlockSpec((D, 1), lambda n: (0, 0)),
            pl.BlockSpec((K, D), lambda n: (0, 0)),
            pl.BlockSpec((D, P_PAD), lambda n: (0, 0)),
            pl.BlockSpec((1, P_PAD), lambda n: (0, 0)),
        ],
        out_specs=[
            pl.BlockSpec((1, 1, N_TILE), lambda n: (n, 0, 0)),
            pl.BlockSpec((1, 1), lambda n: (0, 0),
                         memory_space=pltpu.SMEM),
            pl.BlockSpec((K, P_PAD), lambda n: (0, 0)),
        ],
        out_shape=[
            jax.ShapeDtypeStruct((N_GRID, 1, N_TILE), jnp.int32),
            jax.ShapeDtypeStruct((1, 1), jnp.float32),
            jax.ShapeDtypeStruct((K, P_PAD), jnp.float32),
        ],
        scratch_shapes=[
            pltpu.VMEM((K_GRID, K_TILE, 1), jnp.float32),
        ],
    )(x5, WeT, enc_b, codebook, Wd, bias_patch)


@functools.lru_cache(maxsize=1)
def _make_sc_gather():
    info = plsc.get_sparse_core_info()
    nw = info.num_cores * info.num_subcores
    b_per_w = N // nw
    mesh = plsc.VectorSubcoreMesh(core_axis_name="c", subcore_axis_name="s")

    @functools.partial(
        pl.kernel, mesh=mesh,
        out_type=jax.ShapeDtypeStruct((N, P_PAD), jnp.float32),
        scratch_types=[
            pltpu.VMEM((b_per_w,), jnp.int32),
            pltpu.VMEM((b_per_w, P_PAD), jnp.float32),
            pltpu.SemaphoreType.DMA,
        ],
    )
    def gather_kernel(table_hbm, idx_hbm, out_hbm, idx_v, rows_v, sem):
        wid = lax.axis_index("s") * info.num_cores + lax.axis_index("c")
        base = wid * b_per_w
        pltpu.sync_copy(idx_hbm.at[pl.ds(base, b_per_w)], idx_v)
        pltpu.async_copy(table_hbm.at[idx_v], rows_v, sem).wait()
        pltpu.sync_copy(rows_v, out_hbm.at[pl.ds(base, b_per_w)])

    return gather_kernel


def _unpatchify_body(r_ref, o_ref):
    rp = r_ref[...][:, :P]
    o_ref[0, :, 0] = (rp.reshape(16, 32, 3, 4, 4)
                        .transpose(2, 0, 3, 1, 4)
                        .reshape(3, 64, 128))


def _unpatchify(rp):
    return pl.pallas_call(
        _unpatchify_body,
        grid=(N_GRID,),
        in_specs=[pl.BlockSpec((N_TILE, P_PAD), lambda n: (n, 0))],
        out_specs=pl.BlockSpec((1, 3, 1, 64, 128),
                               lambda n: (n // 2, 0, n % 2, 0, 0)),
        out_shape=jax.ShapeDtypeStruct((4, 3, 2, 64, 128), jnp.float32),
    )(rp)


def kernel(x, enc_W, enc_b, codebook, dec_W, dec_b):
    B = x.shape[0]
    x5 = x.reshape(B, 3, 2, 64, 128)
    WeT = enc_W.reshape(D, P)
    Wd = dec_W[:, :, ::-1, ::-1].transpose(1, 0, 2, 3).reshape(D, P)
    Wd = jnp.concatenate([Wd, jnp.zeros((D, P_PAD - P), Wd.dtype)], axis=1)
    bias_patch = jnp.concatenate(
        [jnp.repeat(dec_b, 16), jnp.zeros((P_PAD - P,), dec_b.dtype)]
    ).reshape(1, P_PAD)

    idx, loss, cw = _encode_quantize(x5, WeT, enc_b.reshape(D, 1),
                                     codebook, Wd, bias_patch)
    recon_p = _make_sc_gather()(cw, idx.reshape(N))
    recon = _unpatchify(recon_p).reshape(B, 3, 128, 128)
    return recon, loss[0, 0]


# speedup vs baseline: 2.1243x; 1.0676x over previous
import functools

import jax
import jax.numpy as jnp
from jax import lax
from jax.experimental import pallas as pl
from jax.experimental.pallas import tpu as pltpu
from jax.experimental.pallas import tpu_sc as plsc

N = 4096
D = 256
K = 8192
P = 48
P_PAD = 128

N_TILE = 512
K_TILE = 512
N_GRID = N // N_TILE
K_GRID = K // K_TILE
SUB_UNROLL = 16


def _encode_quantize_body(x_ref, wet_ref, eb_ref, cb_ref, wd_ref, bias_ref,
                          idx_ref, loss_ref, cw_ref, cnh_ref):
    n = pl.program_id(0)

    @pl.when(n == 0)
    def _once():
        for i in range(K_GRID):
            cb_i = cb_ref[pl.ds(i * K_TILE, K_TILE), :]
            cnh_ref[i] = 0.5 * jnp.sum(cb_i * cb_i, axis=1, keepdims=True)
        cw_ref[...] = (jnp.dot(cb_ref[...], wd_ref[...],
                               preferred_element_type=jnp.float32)
                       + bias_ref[...])

    xb = x_ref[0, :, 0]
    pT = (xb.reshape(3, 16, 4, 32, 4)
            .transpose(0, 2, 4, 1, 3)
            .reshape(P, N_TILE))
    zT = jnp.dot(wet_ref[...], pT, preferred_element_type=jnp.float32)
    zT = jnp.maximum(zT + eb_ref[...], 0.0)

    def sub(i, carry):
        maxval, argid = carry
        off = i * K_TILE
        cb_i = cb_ref[pl.ds(off, K_TILE), :]
        tT = (jnp.dot(cb_i, zT, preferred_element_type=jnp.float32)
              - cnh_ref[i])
        cm = jnp.max(tT, axis=0, keepdims=True)
        iota = lax.broadcasted_iota(jnp.int32, tT.shape, 0)
        li = (jnp.min(jnp.where(tT == cm, iota, K_TILE),
                      axis=0, keepdims=True) + off)
        better = cm > maxval
        return (jnp.where(better, cm, maxval),
                jnp.where(better, li, argid))

    maxval0 = jnp.full((1, N_TILE), -jnp.inf, dtype=jnp.float32)
    argid0 = jnp.zeros((1, N_TILE), dtype=jnp.int32)
    maxval, argid = lax.fori_loop(0, K_GRID, sub, (maxval0, argid0),
                                  unroll=SUB_UNROLL)

    idx_ref[0] = argid
    part = jnp.sum(zT * zT) - 2.0 * jnp.sum(maxval)
    prev = jnp.where(n == 0, 0.0, loss_ref[0, 0])
    tot = prev + part
    scale = jnp.where(n == N_GRID - 1, 1.25 / (N * D), 1.0)
    loss_ref[0, 0] = tot * scale


def _encode_quantize(x5, WeT, enc_b, codebook, Wd, bias_patch):
    return pl.pallas_call(
        _encode_quantize_body,
        grid=(N_GRID,),
        in_specs=[
            pl.BlockSpec((1, 3, 1, 64, 128),
                         lambda n: (n // 2, 0, n % 2, 0, 0)),
            pl.BlockSpec((D, P), lambda n: (0, 0)),
            pl.BlockSpec((D, 1), lambda n: (0, 0)),
            pl.BlockSpec((K, D), lambda n: (0, 0)),
            pl.BlockSpec((D, P_PAD), lambda n: (0, 0)),
            pl.BlockSpec((1, P_PAD), lambda n: (0, 0)),
        ],
        out_specs=[
            pl.BlockSpec((1, 1, N_TILE), lambda n: (n, 0, 0)),
            pl.BlockSpec((1, 1), lambda n: (0, 0),
                         memory_space=pltpu.SMEM),
            pl.BlockSpec((K, P_PAD), lambda n: (0, 0)),
        ],
        out_shape=[
            jax.ShapeDtypeStruct((N_GRID, 1, N_TILE), jnp.int32),
            jax.ShapeDtypeStruct((1, 1), jnp.float32),
            jax.ShapeDtypeStruct((K, P_PAD), jnp.float32),
        ],
        scratch_shapes=[
            pltpu.VMEM((K_GRID, K_TILE, 1), jnp.float32),
        ],
    )(x5, WeT, enc_b, codebook, Wd, bias_patch)


@functools.lru_cache(maxsize=1)
def _make_sc_gather():
    info = plsc.get_sparse_core_info()
    nw = info.num_cores * info.num_subcores
    b_per_w = N // nw
    mesh = plsc.VectorSubcoreMesh(core_axis_name="c", subcore_axis_name="s")

    @functools.partial(
        pl.kernel, mesh=mesh,
        out_type=jax.ShapeDtypeStruct((N, P_PAD), jnp.float32),
        scratch_types=[
            pltpu.VMEM((b_per_w,), jnp.int32),
            pltpu.VMEM((b_per_w, P_PAD), jnp.float32),
            pltpu.SemaphoreType.DMA,
        ],
    )
    def gather_kernel(table_hbm, idx_hbm, out_hbm, idx_v, rows_v, sem):
        wid = lax.axis_index("s") * info.num_cores + lax.axis_index("c")
        base = wid * b_per_w
        pltpu.sync_copy(idx_hbm.at[pl.ds(base, b_per_w)], idx_v)
        pltpu.async_copy(table_hbm.at[idx_v], rows_v, sem).wait()
        pltpu.sync_copy(rows_v, out_hbm.at[pl.ds(base, b_per_w)])

    return gather_kernel


def _unpatchify_body(r_ref, o_ref):
    rp = r_ref[...][:, :P]
    o_ref[0, :, 0] = (rp.reshape(16, 32, 3, 4, 4)
                        .transpose(2, 0, 3, 1, 4)
                        .reshape(3, 64, 128))


def _unpatchify(rp):
    return pl.pallas_call(
        _unpatchify_body,
        grid=(N_GRID,),
        in_specs=[pl.BlockSpec((N_TILE, P_PAD), lambda n: (n, 0))],
        out_specs=pl.BlockSpec((1, 3, 1, 64, 128),
                               lambda n: (n // 2, 0, n % 2, 0, 0)),
        out_shape=jax.ShapeDtypeStruct((4, 3, 2, 64, 128), jnp.float32),
    )(rp)


def kernel(x, enc_W, enc_b, codebook, dec_W, dec_b):
    B = x.shape[0]
    x5 = x.reshape(B, 3, 2, 64, 128)
    WeT = enc_W.reshape(D, P)
    Wd = dec_W[:, :, ::-1, ::-1].transpose(1, 0, 2, 3).reshape(D, P)
    Wd = jnp.concatenate([Wd, jnp.zeros((D, P_PAD - P), Wd.dtype)], axis=1)
    bias_patch = jnp.concatenate(
        [jnp.repeat(dec_b, 16), jnp.zeros((P_PAD - P,), dec_b.dtype)]
    ).reshape(1, P_PAD)

    idx, loss, cw = _encode_quantize(x5, WeT, enc_b.reshape(D, 1),
                                     codebook, Wd, bias_patch)
    recon_p = _make_sc_gather()(cw, idx.reshape(N))
    recon = _unpatchify(recon_p).reshape(B, 3, 128, 128)
    return recon, loss[0, 0]
